# Initial kernel scaffold; baseline (speedup 1.0000x reference)
import jax
import jax.numpy as jnp
from jax.experimental import pallas as pl


def kernel(cls_out_0, cls_out_1, cls_out_2, cls_out_3, cls_out_4, box_out_0, box_out_1, box_out_2, box_out_3, box_out_4, img_scale, anchor_boxes):
    raise NotImplementedError("write your pallas kernel here")


# speedup vs baseline: 40.7704x; 40.7704x over previous
import functools
import jax
import jax.numpy as jnp
from jax.experimental import pallas as pl

N_CLASSES = 90
N_ANCH = 9
FEATS = [64, 32, 16, 8, 4]
HWS = [f * f for f in FEATS]
TOT_HW = sum(HWS)
N_KEEP = 5120
N_DET = 100
IMG = 512.0


def _classmax_body(cls_ref, mx_ref, am_ref):
    x = cls_ref[0]
    r = x.reshape(N_ANCH, N_CLASSES, x.shape[-1])
    m = jnp.max(r, axis=1)
    cio = jax.lax.broadcasted_iota(jnp.int32, r.shape, 1)
    am = jnp.min(jnp.where(r == m[:, None, :], cio, N_CLASSES), axis=1)
    mx_ref[0] = m
    am_ref[0] = am


def _classmax_level(cls, ck):
    b, c, hw = cls.shape
    grid = (b, hw // ck)
    return pl.pallas_call(
        _classmax_body,
        grid=grid,
        in_specs=[pl.BlockSpec((1, c, ck), lambda i, j: (i, 0, j))],
        out_specs=[pl.BlockSpec((1, N_ANCH, ck), lambda i, j: (i, 0, j)),
                   pl.BlockSpec((1, N_ANCH, ck), lambda i, j: (i, 0, j))],
        out_shape=[jax.ShapeDtypeStruct((b, N_ANCH, hw), jnp.float32),
                   jax.ShapeDtypeStruct((b, N_ANCH, hw), jnp.int32)],
    )(cls)


def _nms_body(ty_r, tx_r, th_r, tw_r, ay1_r, ax1_r, ay2_r, ax2_r,
              lg_r, cl_r, scale_r, out_r):
    ay1, ax1, ay2, ax2 = ay1_r[...], ax1_r[...], ay2_r[...], ax2_r[...]
    ha = ay2 - ay1
    wa = ax2 - ax1
    yca = (ay1 + ay2) * 0.5
    xca = (ax1 + ax2) * 0.5
    w = jnp.exp(tw_r[...]) * wa
    h = jnp.exp(th_r[...]) * ha
    yc = ty_r[...] * ha + yca
    xc = tx_r[...] * wa + xca
    y1 = jnp.clip(yc - h * 0.5, 0.0, IMG)
    x1 = jnp.clip(xc - w * 0.5, 0.0, IMG)
    y2 = jnp.clip(yc + h * 0.5, 0.0, IMG)
    x2 = jnp.clip(xc + w * 0.5, 0.0, IMG)
    areas = (y2 - y1) * (x2 - x1)
    cls = cl_r[...]
    scale = scale_r[...]
    sc0 = 1.0 / (1.0 + jnp.exp(-lg_r[...]))
    lin = jax.lax.broadcasted_iota(jnp.int32, sc0.shape, 1)

    def step(i, sc):
        m = jnp.max(sc, axis=1, keepdims=True)
        isel = jnp.min(jnp.where(sc == m, lin, jnp.int32(1 << 30)),
                       axis=1, keepdims=True)
        one = (lin == isel)
        onef = one.astype(jnp.float32)
        by1 = jnp.sum(y1 * onef, axis=1, keepdims=True)
        bx1 = jnp.sum(x1 * onef, axis=1, keepdims=True)
        by2 = jnp.sum(y2 * onef, axis=1, keepdims=True)
        bx2 = jnp.sum(x2 * onef, axis=1, keepdims=True)
        bcl = jnp.sum(cls * onef, axis=1, keepdims=True)
        yy1 = jnp.maximum(by1, y1)
        xx1 = jnp.maximum(bx1, x1)
        yy2 = jnp.minimum(by2, y2)
        xx2 = jnp.minimum(bx2, x2)
        inter = jnp.maximum(yy2 - yy1, 0.0) * jnp.maximum(xx2 - xx1, 0.0)
        area0 = (by2 - by1) * (bx2 - bx1)
        iou = inter / (area0 + areas - inter + 1e-8)
        sc = jnp.where(jnp.logical_or(iou > 0.5, one), -1e10, sc)
        det = jnp.concatenate(
            [bx1 * scale, by1 * scale, bx2 * scale, by2 * scale, m, bcl],
            axis=1)
        out_r[:, i, :] = det
        return sc

    jax.lax.fori_loop(0, N_DET, step, sc0)


def _nms(ty, tx, th, tw, ay1, ax1, ay2, ax2, lg, cl, scale):
    b = lg.shape[0]
    full = lambda s: pl.BlockSpec(s, lambda: tuple(0 for _ in s))
    pln = (b, N_KEEP)
    return pl.pallas_call(
        _nms_body,
        grid=(),
        in_specs=[full(pln)] * 10 + [full((b, 1))],
        out_specs=full((b, N_DET, 6)),
        out_shape=jax.ShapeDtypeStruct((b, N_DET, 6), jnp.float32),
    )(ty, tx, th, tw, ay1, ax1, ay2, ax2, lg, cl, scale)


def kernel(cls_out_0, cls_out_1, cls_out_2, cls_out_3, cls_out_4,
           box_out_0, box_out_1, box_out_2, box_out_3, box_out_4,
           img_scale, anchor_boxes):
    cls_outs = [cls_out_0, cls_out_1, cls_out_2, cls_out_3, cls_out_4]
    box_outs = [box_out_0, box_out_1, box_out_2, box_out_3, box_out_4]
    b = cls_out_0.shape[0]

    mxs, ams = [], []
    for i, (cls, hw) in enumerate(zip(cls_outs, HWS)):
        ck = min(hw, 1024)
        mx, am = _classmax_level(cls.reshape(b, N_ANCH * N_CLASSES, hw), ck)
        mxs.append(mx)
        ams.append(am)
    mx = jnp.concatenate(mxs, axis=2).transpose(0, 2, 1).reshape(
        b, N_ANCH * TOT_HW)
    am = jnp.concatenate(ams, axis=2).transpose(0, 2, 1).reshape(
        b, N_ANCH * TOT_HW)

    lg, ridx = jax.lax.top_k(mx, N_KEEP)

    cl = (jnp.take_along_axis(am, ridx, axis=1).astype(jnp.float32) + 1.0)

    anc = jnp.take(anchor_boxes, ridx, axis=0)
    ay1, ax1, ay2, ax2 = (anc[..., j] for j in range(4))

    box_all = jnp.concatenate(
        [t.reshape(b, N_ANCH * 4, hw) for t, hw in zip(box_outs, HWS)],
        axis=2).reshape(b, N_ANCH, 4, TOT_HW)
    flatb = box_all.transpose(0, 2, 3, 1).reshape(b, 4, N_ANCH * TOT_HW)
    rel = jnp.take_along_axis(flatb, ridx[:, None, :], axis=2)
    ty, tx, th, tw = (rel[:, j, :] for j in range(4))

    return _nms(ty, tx, th, tw, ay1, ax1, ay2, ax2, lg, cl,
                img_scale[:, None])
